# baseline (speedup 1.0000x reference)
import functools

import jax
import jax.numpy as jnp
from jax import lax
from jax.experimental import pallas as pl
from jax.experimental.pallas import tpu as pltpu
from jax.experimental.pallas import tpu_sc as plsc

BATCH, SEQ, EMBED = 32, 2048, 32
VOCAB = 1000000
_LANES = 16

_info = plsc.get_sparse_core_info()
_NC, _NS = _info.num_cores, _info.num_subcores

CHUNK = 256
_ISUB = 128
_NSTR = CHUNK // _ISUB


def _emb_body(x4_hbm, sub_hbm, t128_hbm, pos_hbm, out_hbm,
              idx_v, sub_v, g_v, pv_v, res_v, gsem, psem):
    w = lax.axis_index("s") * _NC + lax.axis_index("c")

    for c in range(SEQ // CHUNK):
        off = c * CHUNK
        base = w * SEQ + off
        pltpu.sync_copy(x4_hbm.at[pl.ds(base, CHUNK)], idx_v)
        pltpu.sync_copy(sub_hbm.at[pl.ds(base, CHUNK)],
                        sub_v.at[pl.ds(0, CHUNK)])
        pcp = pltpu.async_copy(pos_hbm.at[pl.ds(off, CHUNK)], pv_v, psem)
        cps = []
        for k in range(_NSTR):
            cps.append(pltpu.async_copy(
                t128_hbm.at[idx_v.at[pl.ds(k * _ISUB, _ISUB)]],
                g_v.at[pl.ds(k * _ISUB, _ISUB)],
                gsem))
        for cp in cps:
            cp.wait()
        pcp.wait()

        def pick(j, carry):
            o = sub_v[pl.ds(j, _LANES)][0]
            lo = pl.ds(0, _LANES)
            hi = pl.ds(_LANES, _LANES)
            res_v[j, lo] = g_v[j, pl.ds(o, _LANES)] + pv_v[j, lo]
            res_v[j, hi] = g_v[j, pl.ds(o + _LANES, _LANES)] + pv_v[j, hi]
            return carry

        lax.fori_loop(0, CHUNK, pick, 0)

        pltpu.sync_copy(res_v, out_hbm.at[w, pl.ds(off, CHUNK)])


_mesh = plsc.VectorSubcoreMesh(core_axis_name="c", subcore_axis_name="s")

_emb = functools.partial(
    pl.kernel,
    mesh=_mesh,
    out_type=jax.ShapeDtypeStruct((BATCH, SEQ, EMBED), jnp.float32),
    compiler_params=pltpu.CompilerParams(use_tc_tiling_on_sc=False),
    scratch_types=[
        pltpu.VMEM((CHUNK,), jnp.int32),
        pltpu.VMEM((CHUNK + _LANES,), jnp.int32),
        pltpu.VMEM((CHUNK, 4 * EMBED), jnp.float32),
        pltpu.VMEM((CHUNK, EMBED), jnp.float32),
        pltpu.VMEM((CHUNK, EMBED), jnp.float32),
        pltpu.SemaphoreType.DMA,
        pltpu.SemaphoreType.DMA,
    ],
)(_emb_body)


def kernel(x, token_table, pos_table):
    t128 = token_table.reshape(VOCAB // 4, 4 * EMBED)
    xf = x.astype(jnp.int32).reshape(BATCH * SEQ)
    x4 = xf >> 2
    sub = (xf & 3) * EMBED
    return _emb(x4, sub, t128, pos_table)


# speedup vs baseline: 1.1143x; 1.1143x over previous
import functools

import jax
import jax.numpy as jnp
from jax import lax
from jax.experimental import pallas as pl
from jax.experimental.pallas import tpu as pltpu
from jax.experimental.pallas import tpu_sc as plsc

BATCH, SEQ, EMBED = 32, 2048, 32
_LANES = 16

_info = plsc.get_sparse_core_info()
_NC, _NS = _info.num_cores, _info.num_subcores
_NW = _NC * _NS

CHUNK = 1024
NCHUNK = SEQ // CHUNK
GSUB = 128
NG = CHUNK // GSUB


def _emb_body(x_hbm, tok_hbm, pos_hbm, out_hbm, idx_v, buf_v, pos_v,
              gsem, psem):
    w = lax.axis_index("s") * _NC + lax.axis_index("c")
    pltpu.sync_copy(x_hbm.at[w], idx_v)

    for c in range(NCHUNK):
        off = c * CHUNK
        pcp = pltpu.async_copy(pos_hbm.at[pl.ds(off, CHUNK)], pos_v, psem)
        cps = []
        for j in range(NG):
            cps.append(pltpu.async_copy(
                tok_hbm.at[idx_v.at[pl.ds(off + j * GSUB, GSUB)]],
                buf_v.at[pl.ds(j * GSUB, GSUB)],
                gsem))
        for cp in cps:
            cp.wait()
        pcp.wait()

        def add_row(i, carry):
            lo = pl.ds(0, _LANES)
            hi = pl.ds(_LANES, _LANES)
            plsc.addupdate(buf_v.at[i, lo], pos_v[i, lo])
            plsc.addupdate(buf_v.at[i, hi], pos_v[i, hi])
            return carry

        lax.fori_loop(0, CHUNK, add_row, 0)

        pltpu.sync_copy(buf_v, out_hbm.at[w, pl.ds(off, CHUNK)])


_mesh = plsc.VectorSubcoreMesh(core_axis_name="c", subcore_axis_name="s")

_emb = functools.partial(
    pl.kernel,
    mesh=_mesh,
    out_type=jax.ShapeDtypeStruct((BATCH, SEQ, EMBED), jnp.float32),
    compiler_params=pltpu.CompilerParams(use_tc_tiling_on_sc=False),
    scratch_types=[
        pltpu.VMEM((SEQ,), jnp.int32),
        pltpu.VMEM((CHUNK, EMBED), jnp.float32),
        pltpu.VMEM((CHUNK, EMBED), jnp.float32),
        pltpu.SemaphoreType.DMA,
        pltpu.SemaphoreType.DMA,
    ],
)(_emb_body)


def kernel(x, token_table, pos_table):
    return _emb(x.astype(jnp.int32), token_table, pos_table)
